# baseline (speedup 1.0000x reference)
import functools

import jax
import jax.numpy as jnp
from jax import lax
from jax.experimental import pallas as pl
from jax.experimental.pallas import tpu as pltpu
from jax.experimental.pallas import tpu_sc as plsc

NC = 2
NS = 16
NW = NC * NS
LW = 16
K = 128
DW = 128
BR = 1024

def _mesh():
    return plsc.VectorSubcoreMesh(core_axis_name="c", subcore_axis_name="s",
                                  num_cores=NC, num_subcores=NS)


def _make_deg_kernel(npad, nb):
    rt = npad // NS

    @functools.partial(
        pl.kernel,
        out_type=jax.ShapeDtypeStruct((NC, npad, DW), jnp.float32),
        mesh=_mesh(),
        scratch_types=[
            pltpu.VMEM((nb, K), jnp.int32),
            pltpu.VMEM((K, DW), jnp.float32),
            pltpu.VMEM_SHARED((npad, DW), jnp.float32),
        ],
    )
    def deg_kernel(dst_hbm, out_hbm, dst_v, ones_v, hist_sh):
        c = lax.axis_index("c")
        s = lax.axis_index("s")
        wid = c * NS + s
        pltpu.sync_copy(dst_hbm.at[wid], dst_v)

        def _zrow(i, carry):
            for k in range(DW // LW):
                ones_v[i, pl.ds(k * LW, LW)] = jnp.zeros((LW,), jnp.float32)
            return carry

        lax.fori_loop(0, K, _zrow, 0)
        for r in range(rt // K):
            pltpu.sync_copy(ones_v, hist_sh.at[pl.ds(s * rt + r * K, K)])

        def _orow(i, carry):
            for k in range(DW // LW):
                ones_v[i, pl.ds(k * LW, LW)] = jnp.ones((LW,), jnp.float32)
            return carry

        lax.fori_loop(0, K, _orow, 0)
        plsc.subcore_barrier()

        def _blk(j, carry):
            pltpu.sync_copy(ones_v, hist_sh.at[dst_v.at[j]], add=True)
            return carry

        lax.fori_loop(0, nb, _blk, 0)
        plsc.subcore_barrier()
        pltpu.sync_copy(hist_sh.at[pl.ds(s * rt, rt)],
                        out_hbm.at[c, pl.ds(s * rt, rt)])

    return deg_kernel


CH = 40


def _make_scatter_kernel(npad, nb, d):
    rt = npad // NS

    @functools.partial(
        pl.kernel,
        out_type=jax.ShapeDtypeStruct((NC, npad, d), jnp.float32),
        mesh=_mesh(),
        scratch_types=[
            pltpu.VMEM((CH, K), jnp.int32),
            pltpu.VMEM((CH, K), jnp.int32),
            pltpu.VMEM((K, d), jnp.float32),
            pltpu.VMEM((K, d), jnp.float32),
            pltpu.VMEM_SHARED((npad, d), jnp.float32),
            pltpu.SemaphoreType.DMA,
            pltpu.SemaphoreType.DMA,
        ],
    )
    def scatter_kernel(g_hbm, src_hbm, dst_hbm, out_hbm,
                       src_ib, dst_ib, gb0, gb1, acc_sh, sem0, sem1):
        c = lax.axis_index("c")
        s = lax.axis_index("s")
        wid = c * NS + s

        def _zrow(i, carry):
            for k in range(d // LW):
                gb0[i, pl.ds(k * LW, LW)] = jnp.zeros((LW,), jnp.float32)
            return carry

        lax.fori_loop(0, K, _zrow, 0)
        for r in range(rt // K):
            pltpu.sync_copy(gb0, acc_sh.at[pl.ds(s * rt + r * K, K)])
        plsc.subcore_barrier()

        def _chunk(cidx, carry):
            pltpu.sync_copy(src_hbm.at[wid, pl.ds(cidx * CH, CH)], src_ib)
            pltpu.sync_copy(dst_hbm.at[wid, pl.ds(cidx * CH, CH)], dst_ib)
            pltpu.async_copy(g_hbm.at[src_ib.at[0]], gb0, sem0)

            def _step(it, inner):
                j0 = it * 2
                j1 = j0 + 1
                j2 = j0 + 2
                pltpu.make_async_copy(g_hbm.at[src_ib.at[j0]], gb0, sem0).wait()
                pltpu.async_copy(g_hbm.at[src_ib.at[j1]], gb1, sem1)
                pltpu.sync_copy(gb0, acc_sh.at[dst_ib.at[j0]], add=True)
                pltpu.make_async_copy(g_hbm.at[src_ib.at[j1]], gb1, sem1).wait()

                @pl.when(j2 < CH)
                def _():
                    pltpu.async_copy(g_hbm.at[src_ib.at[j2]], gb0, sem0)

                pltpu.sync_copy(gb1, acc_sh.at[dst_ib.at[j1]], add=True)
                return inner

            lax.fori_loop(0, CH // 2, _step, 0)
            return carry

        lax.fori_loop(0, nb // CH, _chunk, 0)
        plsc.subcore_barrier()
        pltpu.sync_copy(acc_sh.at[pl.ds(s * rt, rt)],
                        out_hbm.at[c, pl.ds(s * rt, rt)])

    return scatter_kernel


def _mm_body(deg_ref, x_ref, w_ref, g_ref, dis_ref):
    dd = deg_ref[...]
    deg = dd[0, :, 0:1] + dd[1, :, 0:1] + 1.0
    dis = lax.rsqrt(deg)
    h = jnp.dot(x_ref[...], w_ref[...], preferred_element_type=jnp.float32)
    g_ref[...] = h * dis
    dis_ref[...] = dis


def _out_body(acc_ref, g_ref, dis_ref, b_ref, a_ref, o_ref):
    aa = acc_ref[...]
    t = (aa[0] + aa[1] + g_ref[...]) * dis_ref[...] + b_ref[...]
    o_ref[...] = jnp.where(t >= 0.0, t, a_ref[...] * t)


def kernel(x, edge_index, W, b, alpha):
    n, d_in = x.shape
    d = W.shape[1]
    e = edge_index.shape[1]

    npad = ((n + BR - 1) // BR) * BR
    nb = -(-e // (NW * K))
    nb = ((nb + CH - 1) // CH) * CH
    epad = NW * nb * K

    x_pad = jnp.zeros((npad, d_in), x.dtype).at[:n].set(x)
    pad = n + jnp.arange(epad - e, dtype=edge_index.dtype) % (npad - n)
    srcp = jnp.concatenate([edge_index[0], pad]).reshape(NW, nb, K)
    dstp = jnp.concatenate([edge_index[1], pad]).reshape(NW, nb, K)

    degp = _make_deg_kernel(npad, nb)(dstp)

    nblocks = npad // BR
    g, dis = pl.pallas_call(
        _mm_body,
        grid=(nblocks,),
        in_specs=[
            pl.BlockSpec((NC, BR, DW), lambda i: (0, i, 0)),
            pl.BlockSpec((BR, d_in), lambda i: (i, 0)),
            pl.BlockSpec((d_in, d), lambda i: (0, 0)),
        ],
        out_specs=[
            pl.BlockSpec((BR, d), lambda i: (i, 0)),
            pl.BlockSpec((BR, 1), lambda i: (i, 0)),
        ],
        out_shape=[
            jax.ShapeDtypeStruct((npad, d), jnp.float32),
            jax.ShapeDtypeStruct((npad, 1), jnp.float32),
        ],
    )(degp, x_pad, W)

    accp = _make_scatter_kernel(npad, nb, d)(g, srcp, dstp)

    out = pl.pallas_call(
        _out_body,
        grid=(nblocks,),
        in_specs=[
            pl.BlockSpec((NC, BR, d), lambda i: (0, i, 0)),
            pl.BlockSpec((BR, d), lambda i: (i, 0)),
            pl.BlockSpec((BR, 1), lambda i: (i, 0)),
            pl.BlockSpec((1, d), lambda i: (0, 0)),
            pl.BlockSpec((1, 1), lambda i: (0, 0)),
        ],
        out_specs=pl.BlockSpec((BR, d), lambda i: (i, 0)),
        out_shape=jax.ShapeDtypeStruct((npad, d), jnp.float32),
    )(accp, g, dis, b.reshape(1, d), alpha.reshape(1, 1))

    return out[:n]


# speedup vs baseline: 1.0211x; 1.0211x over previous
import functools

import jax
import jax.numpy as jnp
from jax import lax
from jax.experimental import pallas as pl
from jax.experimental.pallas import tpu as pltpu
from jax.experimental.pallas import tpu_sc as plsc

NC = 2
NS = 16
NW = NC * NS
LW = 16
K = 128
DW = 128
BR = 1024

def _mesh():
    return plsc.VectorSubcoreMesh(core_axis_name="c", subcore_axis_name="s",
                                  num_cores=NC, num_subcores=NS)


def _make_deg_kernel(npad, nb):
    rt = npad // NS

    @functools.partial(
        pl.kernel,
        out_type=jax.ShapeDtypeStruct((NC, npad, DW), jnp.float32),
        mesh=_mesh(),
        scratch_types=[
            pltpu.VMEM((nb, K), jnp.int32),
            pltpu.VMEM((K, DW), jnp.float32),
            pltpu.VMEM_SHARED((npad, DW), jnp.float32),
        ],
    )
    def deg_kernel(dst_hbm, out_hbm, dst_v, ones_v, hist_sh):
        c = lax.axis_index("c")
        s = lax.axis_index("s")
        wid = c * NS + s
        pltpu.sync_copy(dst_hbm.at[wid], dst_v)

        def _zrow(i, carry):
            for k in range(DW // LW):
                ones_v[i, pl.ds(k * LW, LW)] = jnp.zeros((LW,), jnp.float32)
            return carry

        lax.fori_loop(0, K, _zrow, 0)
        for r in range(rt // K):
            pltpu.sync_copy(ones_v, hist_sh.at[pl.ds(s * rt + r * K, K)])

        def _orow(i, carry):
            for k in range(DW // LW):
                ones_v[i, pl.ds(k * LW, LW)] = jnp.ones((LW,), jnp.float32)
            return carry

        lax.fori_loop(0, K, _orow, 0)
        plsc.subcore_barrier()

        def _blk(j, carry):
            pltpu.sync_copy(ones_v, hist_sh.at[dst_v.at[j]], add=True)
            return carry

        lax.fori_loop(0, nb, _blk, 0)
        plsc.subcore_barrier()
        pltpu.sync_copy(hist_sh.at[pl.ds(s * rt, rt)],
                        out_hbm.at[c, pl.ds(s * rt, rt)])

    return deg_kernel


CH = 40


def _make_scatter_kernel(npad, nb, d):
    rt = npad // NS

    @functools.partial(
        pl.kernel,
        out_type=jax.ShapeDtypeStruct((NC, npad, d), jnp.float32),
        mesh=_mesh(),
        scratch_types=[
            pltpu.VMEM((CH, K), jnp.int32),
            pltpu.VMEM((CH, K), jnp.int32),
            pltpu.VMEM((K, d), jnp.float32),
            pltpu.VMEM((K, d), jnp.float32),
            pltpu.VMEM_SHARED((npad, d), jnp.float32),
            pltpu.SemaphoreType.DMA,
            pltpu.SemaphoreType.DMA,
        ],
    )
    def scatter_kernel(g_hbm, src_hbm, dst_hbm, out_hbm,
                       src_ib, dst_ib, gb0, gb1, acc_sh, sem0, sem1):
        c = lax.axis_index("c")
        s = lax.axis_index("s")
        wid = c * NS + s

        def _zrow(i, carry):
            for k in range(d // LW):
                gb0[i, pl.ds(k * LW, LW)] = jnp.zeros((LW,), jnp.float32)
            return carry

        lax.fori_loop(0, K, _zrow, 0)
        for r in range(rt // K):
            pltpu.sync_copy(gb0, acc_sh.at[pl.ds(s * rt + r * K, K)])
        plsc.subcore_barrier()

        def _chunk(cidx, carry):
            pltpu.sync_copy(src_hbm.at[wid, pl.ds(cidx * CH, CH)], src_ib)
            pltpu.sync_copy(dst_hbm.at[wid, pl.ds(cidx * CH, CH)], dst_ib)
            pltpu.async_copy(g_hbm.at[src_ib.at[0]], gb0, sem0)

            def _step(it, inner):
                j0 = it * 2
                j1 = j0 + 1
                j2 = j0 + 2
                pltpu.make_async_copy(g_hbm.at[src_ib.at[j0]], gb0, sem0).wait()
                pltpu.async_copy(g_hbm.at[src_ib.at[j1]], gb1, sem1)
                pltpu.sync_copy(gb0, acc_sh.at[dst_ib.at[j0]], add=True)
                pltpu.make_async_copy(g_hbm.at[src_ib.at[j1]], gb1, sem1).wait()

                @pl.when(j2 < CH)
                def _():
                    pltpu.async_copy(g_hbm.at[src_ib.at[j2]], gb0, sem0)

                pltpu.sync_copy(gb1, acc_sh.at[dst_ib.at[j1]], add=True)
                return inner

            lax.fori_loop(0, CH // 2, _step, 0)
            return carry

        lax.fori_loop(0, nb // CH, _chunk, 0)
        plsc.subcore_barrier()
        pltpu.sync_copy(acc_sh.at[pl.ds(s * rt, rt)],
                        out_hbm.at[c, pl.ds(s * rt, rt)])

    return scatter_kernel


def _mm_body(deg_ref, x_ref, w_ref, g_ref, dis_ref):
    dd = deg_ref[...]
    deg = dd[0, :, 0:1] + dd[1, :, 0:1] + 1.0
    dis = lax.rsqrt(deg)
    h = jnp.dot(x_ref[...], w_ref[...], preferred_element_type=jnp.float32)
    g_ref[...] = h * dis
    dis_ref[...] = dis


def _out_body(acc_ref, g_ref, dis_ref, b_ref, a_ref, o_ref):
    aa = acc_ref[...]
    t = (aa[0] + aa[1] + g_ref[...]) * dis_ref[...] + b_ref[...]
    o_ref[...] = jnp.where(t >= 0.0, t, a_ref[...] * t)


def kernel(x, edge_index, W, b, alpha):
    n, d_in = x.shape
    d = W.shape[1]
    e = edge_index.shape[1]

    npad = ((n + BR - 1) // BR) * BR
    nb = -(-e // (NW * K))
    nb = ((nb + CH - 1) // CH) * CH
    epad = NW * nb * K

    pad = n + jnp.arange(epad - e, dtype=edge_index.dtype) % (npad - n)
    srcp = jnp.concatenate([edge_index[0], pad]).reshape(NW, nb, K)
    dstp = jnp.concatenate([edge_index[1], pad]).reshape(NW, nb, K)

    degp = _make_deg_kernel(npad, nb)(dstp)

    if n % 80 == 0:
        bn, xin, rows = n // 10, x, n
    else:
        bn, rows = BR, npad
        xin = jnp.zeros((npad, d_in), x.dtype).at[:n].set(x)
    nblocks = rows // bn

    g, dis = pl.pallas_call(
        _mm_body,
        grid=(nblocks,),
        in_specs=[
            pl.BlockSpec((NC, bn, DW), lambda i: (0, i, 0)),
            pl.BlockSpec((bn, d_in), lambda i: (i, 0)),
            pl.BlockSpec((d_in, d), lambda i: (0, 0)),
        ],
        out_specs=[
            pl.BlockSpec((bn, d), lambda i: (i, 0)),
            pl.BlockSpec((bn, 1), lambda i: (i, 0)),
        ],
        out_shape=[
            jax.ShapeDtypeStruct((npad, d), jnp.float32),
            jax.ShapeDtypeStruct((npad, 1), jnp.float32),
        ],
    )(degp, xin, W)

    accp = _make_scatter_kernel(npad, nb, d)(g, srcp, dstp)

    out = pl.pallas_call(
        _out_body,
        grid=(nblocks,),
        in_specs=[
            pl.BlockSpec((NC, bn, d), lambda i: (0, i, 0)),
            pl.BlockSpec((bn, d), lambda i: (i, 0)),
            pl.BlockSpec((bn, 1), lambda i: (i, 0)),
            pl.BlockSpec((1, d), lambda i: (0, 0)),
            pl.BlockSpec((1, 1), lambda i: (0, 0)),
        ],
        out_specs=pl.BlockSpec((bn, d), lambda i: (i, 0)),
        out_shape=jax.ShapeDtypeStruct((rows, d), jnp.float32),
    )(accp, g, dis, b.reshape(1, d), alpha.reshape(1, 1))

    return out[:n]


# speedup vs baseline: 1.0230x; 1.0019x over previous
import functools

import jax
import jax.numpy as jnp
from jax import lax
from jax.experimental import pallas as pl
from jax.experimental.pallas import tpu as pltpu
from jax.experimental.pallas import tpu_sc as plsc

NC = 2
NS = 16
NW = NC * NS
LW = 16
K = 128
DW = 128
BR = 1024

def _mesh():
    return plsc.VectorSubcoreMesh(core_axis_name="c", subcore_axis_name="s",
                                  num_cores=NC, num_subcores=NS)


def _make_deg_kernel(npad, nb):
    rt = npad // NS

    @functools.partial(
        pl.kernel,
        out_type=jax.ShapeDtypeStruct((NC, npad, DW), jnp.float32),
        mesh=_mesh(),
        scratch_types=[
            pltpu.VMEM((nb, K), jnp.int32),
            pltpu.VMEM((K, DW), jnp.float32),
            pltpu.VMEM_SHARED((npad, DW), jnp.float32),
            pltpu.SemaphoreType.DMA,
        ],
    )
    def deg_kernel(dst_hbm, out_hbm, dst_v, ones_v, hist_sh, dsem):
        c = lax.axis_index("c")
        s = lax.axis_index("s")
        wid = c * NS + s
        pltpu.sync_copy(dst_hbm.at[wid], dst_v)

        def _zrow(i, carry):
            for k in range(DW // LW):
                ones_v[i, pl.ds(k * LW, LW)] = jnp.zeros((LW,), jnp.float32)
            return carry

        lax.fori_loop(0, K, _zrow, 0)
        for r in range(rt // K):
            pltpu.sync_copy(ones_v, hist_sh.at[pl.ds(s * rt + r * K, K)])

        def _orow(i, carry):
            for k in range(DW // LW):
                ones_v[i, pl.ds(k * LW, LW)] = jnp.ones((LW,), jnp.float32)
            return carry

        lax.fori_loop(0, K, _orow, 0)
        plsc.subcore_barrier()

        def _blk(j, carry):
            pltpu.async_copy(ones_v, hist_sh.at[dst_v.at[j]], dsem, add=True)
            return carry

        lax.fori_loop(0, nb, _blk, 0)

        def _drain(j, carry):
            pltpu.make_async_copy(ones_v, hist_sh.at[dst_v.at[j]],
                                  dsem).wait()
            return carry

        lax.fori_loop(0, nb, _drain, 0)
        plsc.subcore_barrier()
        pltpu.sync_copy(hist_sh.at[pl.ds(s * rt, rt)],
                        out_hbm.at[c, pl.ds(s * rt, rt)])

    return deg_kernel


CH = 40


def _make_scatter_kernel(npad, nb, d):
    rt = npad // NS

    @functools.partial(
        pl.kernel,
        out_type=jax.ShapeDtypeStruct((NC, npad, d), jnp.float32),
        mesh=_mesh(),
        scratch_types=[
            pltpu.VMEM((CH, K), jnp.int32),
            pltpu.VMEM((CH, K), jnp.int32),
            pltpu.VMEM((K, d), jnp.float32),
            pltpu.VMEM((K, d), jnp.float32),
            pltpu.VMEM_SHARED((npad, d), jnp.float32),
            pltpu.SemaphoreType.DMA,
            pltpu.SemaphoreType.DMA,
        ],
    )
    def scatter_kernel(g_hbm, src_hbm, dst_hbm, out_hbm,
                       src_ib, dst_ib, gb0, gb1, acc_sh, sem0, sem1):
        c = lax.axis_index("c")
        s = lax.axis_index("s")
        wid = c * NS + s

        def _zrow(i, carry):
            for k in range(d // LW):
                gb0[i, pl.ds(k * LW, LW)] = jnp.zeros((LW,), jnp.float32)
            return carry

        lax.fori_loop(0, K, _zrow, 0)
        for r in range(rt // K):
            pltpu.sync_copy(gb0, acc_sh.at[pl.ds(s * rt + r * K, K)])
        plsc.subcore_barrier()

        def _chunk(cidx, carry):
            pltpu.sync_copy(src_hbm.at[wid, pl.ds(cidx * CH, CH)], src_ib)
            pltpu.sync_copy(dst_hbm.at[wid, pl.ds(cidx * CH, CH)], dst_ib)
            pltpu.async_copy(g_hbm.at[src_ib.at[0]], gb0, sem0)

            def _step(it, inner):
                j0 = it * 2
                j1 = j0 + 1
                j2 = j0 + 2
                pltpu.make_async_copy(g_hbm.at[src_ib.at[j0]], gb0, sem0).wait()
                pltpu.async_copy(g_hbm.at[src_ib.at[j1]], gb1, sem1)
                pltpu.sync_copy(gb0, acc_sh.at[dst_ib.at[j0]], add=True)
                pltpu.make_async_copy(g_hbm.at[src_ib.at[j1]], gb1, sem1).wait()

                @pl.when(j2 < CH)
                def _():
                    pltpu.async_copy(g_hbm.at[src_ib.at[j2]], gb0, sem0)

                pltpu.sync_copy(gb1, acc_sh.at[dst_ib.at[j1]], add=True)
                return inner

            lax.fori_loop(0, CH // 2, _step, 0)
            return carry

        lax.fori_loop(0, nb // CH, _chunk, 0)
        plsc.subcore_barrier()
        pltpu.sync_copy(acc_sh.at[pl.ds(s * rt, rt)],
                        out_hbm.at[c, pl.ds(s * rt, rt)])

    return scatter_kernel


def _mm_body(deg_ref, x_ref, w_ref, g_ref, dis_ref):
    dd = deg_ref[...]
    deg = dd[0, :, 0:1] + dd[1, :, 0:1] + 1.0
    dis = lax.rsqrt(deg)
    h = jnp.dot(x_ref[...], w_ref[...], preferred_element_type=jnp.float32)
    g_ref[...] = h * dis
    dis_ref[...] = dis


def _out_body(acc_ref, g_ref, dis_ref, b_ref, a_ref, o_ref):
    aa = acc_ref[...]
    t = (aa[0] + aa[1] + g_ref[...]) * dis_ref[...] + b_ref[...]
    o_ref[...] = jnp.where(t >= 0.0, t, a_ref[...] * t)


def kernel(x, edge_index, W, b, alpha):
    n, d_in = x.shape
    d = W.shape[1]
    e = edge_index.shape[1]

    npad = ((n + BR - 1) // BR) * BR
    nb = -(-e // (NW * K))
    nb = ((nb + CH - 1) // CH) * CH
    epad = NW * nb * K

    pad = n + jnp.arange(epad - e, dtype=edge_index.dtype) % (npad - n)
    srcp = jnp.concatenate([edge_index[0], pad]).reshape(NW, nb, K)
    dstp = jnp.concatenate([edge_index[1], pad]).reshape(NW, nb, K)

    degp = _make_deg_kernel(npad, nb)(dstp)

    if n % 80 == 0:
        bn, xin, rows = n // 10, x, n
    else:
        bn, rows = BR, npad
        xin = jnp.zeros((npad, d_in), x.dtype).at[:n].set(x)
    nblocks = rows // bn

    g, dis = pl.pallas_call(
        _mm_body,
        grid=(nblocks,),
        in_specs=[
            pl.BlockSpec((NC, bn, DW), lambda i: (0, i, 0)),
            pl.BlockSpec((bn, d_in), lambda i: (i, 0)),
            pl.BlockSpec((d_in, d), lambda i: (0, 0)),
        ],
        out_specs=[
            pl.BlockSpec((bn, d), lambda i: (i, 0)),
            pl.BlockSpec((bn, 1), lambda i: (i, 0)),
        ],
        out_shape=[
            jax.ShapeDtypeStruct((npad, d), jnp.float32),
            jax.ShapeDtypeStruct((npad, 1), jnp.float32),
        ],
    )(degp, xin, W)

    accp = _make_scatter_kernel(npad, nb, d)(g, srcp, dstp)

    out = pl.pallas_call(
        _out_body,
        grid=(nblocks,),
        in_specs=[
            pl.BlockSpec((NC, bn, d), lambda i: (0, i, 0)),
            pl.BlockSpec((bn, d), lambda i: (i, 0)),
            pl.BlockSpec((bn, 1), lambda i: (i, 0)),
            pl.BlockSpec((1, d), lambda i: (0, 0)),
            pl.BlockSpec((1, 1), lambda i: (0, 0)),
        ],
        out_specs=pl.BlockSpec((bn, d), lambda i: (i, 0)),
        out_shape=jax.ShapeDtypeStruct((rows, d), jnp.float32),
    )(accp, g, dis, b.reshape(1, d), alpha.reshape(1, 1))

    return out[:n]


# speedup vs baseline: 1.0254x; 1.0023x over previous
import functools

import jax
import jax.numpy as jnp
from jax import lax
from jax.experimental import pallas as pl
from jax.experimental.pallas import tpu as pltpu
from jax.experimental.pallas import tpu_sc as plsc

NC = 2
NS = 16
NW = NC * NS
LW = 16
K = 128
DW = 128
BR = 1024

def _mesh():
    return plsc.VectorSubcoreMesh(core_axis_name="c", subcore_axis_name="s",
                                  num_cores=NC, num_subcores=NS)


def _make_deg_kernel(npad, nb):
    rt = npad // NS

    @functools.partial(
        pl.kernel,
        out_type=jax.ShapeDtypeStruct((NC, npad, DW), jnp.float32),
        mesh=_mesh(),
        scratch_types=[
            pltpu.VMEM((nb, K), jnp.int32),
            pltpu.VMEM((K, DW), jnp.float32),
            pltpu.VMEM_SHARED((npad, DW), jnp.float32),
            pltpu.SemaphoreType.DMA,
        ],
    )
    def deg_kernel(dst_hbm, out_hbm, dst_v, ones_v, hist_sh, dsem):
        c = lax.axis_index("c")
        s = lax.axis_index("s")
        wid = c * NS + s
        pltpu.sync_copy(dst_hbm.at[wid], dst_v)

        def _zrow(i, carry):
            for k in range(DW // LW):
                ones_v[i, pl.ds(k * LW, LW)] = jnp.zeros((LW,), jnp.float32)
            return carry

        lax.fori_loop(0, K, _zrow, 0)
        for r in range(rt // K):
            pltpu.sync_copy(ones_v, hist_sh.at[pl.ds(s * rt + r * K, K)])

        def _orow(i, carry):
            for k in range(DW // LW):
                ones_v[i, pl.ds(k * LW, LW)] = jnp.ones((LW,), jnp.float32)
            return carry

        lax.fori_loop(0, K, _orow, 0)
        plsc.subcore_barrier()

        def _blk(j, carry):
            pltpu.async_copy(ones_v, hist_sh.at[dst_v.at[j]], dsem, add=True)
            return carry

        lax.fori_loop(0, nb, _blk, 0)

        def _drain(j, carry):
            pltpu.make_async_copy(ones_v, hist_sh.at[dst_v.at[j]],
                                  dsem).wait()
            return carry

        lax.fori_loop(0, nb, _drain, 0)
        plsc.subcore_barrier()
        pltpu.sync_copy(hist_sh.at[pl.ds(s * rt, rt)],
                        out_hbm.at[c, pl.ds(s * rt, rt)])

    return deg_kernel


CH = 40


def _make_scatter_kernel(npad, nb, d):
    rt = npad // NS

    @functools.partial(
        pl.kernel,
        out_type=jax.ShapeDtypeStruct((NC, npad, d), jnp.float32),
        mesh=_mesh(),
        scratch_types=[
            pltpu.VMEM((CH, K), jnp.int32),
            pltpu.VMEM((CH, K), jnp.int32),
            pltpu.VMEM((K, d), jnp.float32),
            pltpu.VMEM((K, d), jnp.float32),
            pltpu.VMEM_SHARED((npad, d), jnp.float32),
            pltpu.SemaphoreType.DMA,
            pltpu.SemaphoreType.DMA,
        ],
    )
    def scatter_kernel(g_hbm, src_hbm, dst_hbm, out_hbm,
                       src_ib, dst_ib, gb0, gb1, acc_sh, sem0, sem1):
        c = lax.axis_index("c")
        s = lax.axis_index("s")
        wid = c * NS + s

        def _zrow(i, carry):
            for k in range(d // LW):
                gb0[i, pl.ds(k * LW, LW)] = jnp.zeros((LW,), jnp.float32)
            return carry

        lax.fori_loop(0, K, _zrow, 0)
        for r in range(rt // K):
            pltpu.sync_copy(gb0, acc_sh.at[pl.ds(s * rt + r * K, K)])
        plsc.subcore_barrier()

        def _chunk(cidx, carry):
            pltpu.sync_copy(src_hbm.at[wid, pl.ds(cidx * CH, CH)], src_ib)
            pltpu.sync_copy(dst_hbm.at[wid, pl.ds(cidx * CH, CH)], dst_ib)
            pltpu.async_copy(g_hbm.at[src_ib.at[0]], gb0, sem0)

            def _step(it, inner):
                j0 = it * 2
                j1 = j0 + 1
                j2 = j0 + 2
                pltpu.make_async_copy(g_hbm.at[src_ib.at[j0]], gb0, sem0).wait()
                pltpu.async_copy(g_hbm.at[src_ib.at[j1]], gb1, sem1)
                pltpu.sync_copy(gb0, acc_sh.at[dst_ib.at[j0]], add=True)
                pltpu.make_async_copy(g_hbm.at[src_ib.at[j1]], gb1, sem1).wait()

                @pl.when(j2 < CH)
                def _():
                    pltpu.async_copy(g_hbm.at[src_ib.at[j2]], gb0, sem0)

                pltpu.sync_copy(gb1, acc_sh.at[dst_ib.at[j1]], add=True)
                return inner

            lax.fori_loop(0, CH // 2, _step, 0)
            return carry

        lax.fori_loop(0, nb // CH, _chunk, 0)
        plsc.subcore_barrier()
        pltpu.sync_copy(acc_sh.at[pl.ds(s * rt, rt)],
                        out_hbm.at[c, pl.ds(s * rt, rt)])

    return scatter_kernel


def _mm_body(deg_ref, x_ref, w_ref, g_ref, dis_ref):
    dd = deg_ref[...]
    deg = dd[0, :, 0:1] + dd[1, :, 0:1] + 1.0
    dis = lax.rsqrt(deg)
    h = jnp.dot(x_ref[...], w_ref[...], preferred_element_type=jnp.float32)
    g_ref[...] = h * dis
    dis_ref[...] = dis


def _out_body(acc_ref, g_ref, dis_ref, b_ref, a_ref, o_ref):
    aa = acc_ref[...]
    t = (aa[0] + aa[1] + g_ref[...]) * dis_ref[...] + b_ref[...]
    o_ref[...] = jnp.where(t >= 0.0, t, a_ref[...] * t)


def kernel(x, edge_index, W, b, alpha):
    n, d_in = x.shape
    d = W.shape[1]
    e = edge_index.shape[1]

    npad = ((n + 2048) // 2048) * 2048
    nb = -(-e // (NW * K))
    nb = ((nb + CH - 1) // CH) * CH
    epad = NW * nb * K

    pad = n + jnp.arange(epad - e, dtype=edge_index.dtype) % (npad - n)
    srcp = jnp.concatenate([edge_index[0], pad]).reshape(NW, nb, K)
    dstp = jnp.concatenate([edge_index[1], pad]).reshape(NW, nb, K)

    degp = _make_deg_kernel(npad, nb)(dstp)

    if n % 80 == 0:
        bn, xin, rows = n // 10, x, n
    else:
        bn, rows = BR, npad
        xin = jnp.zeros((npad, d_in), x.dtype).at[:n].set(x)
    nblocks = rows // bn

    g, dis = pl.pallas_call(
        _mm_body,
        grid=(nblocks,),
        in_specs=[
            pl.BlockSpec((NC, bn, DW), lambda i: (0, i, 0)),
            pl.BlockSpec((bn, d_in), lambda i: (i, 0)),
            pl.BlockSpec((d_in, d), lambda i: (0, 0)),
        ],
        out_specs=[
            pl.BlockSpec((bn, d), lambda i: (i, 0)),
            pl.BlockSpec((bn, 1), lambda i: (i, 0)),
        ],
        out_shape=[
            jax.ShapeDtypeStruct((npad, d), jnp.float32),
            jax.ShapeDtypeStruct((npad, 1), jnp.float32),
        ],
    )(degp, xin, W)

    accp = _make_scatter_kernel(npad, nb, d)(g, srcp, dstp)

    out = pl.pallas_call(
        _out_body,
        grid=(nblocks,),
        in_specs=[
            pl.BlockSpec((NC, bn, d), lambda i: (0, i, 0)),
            pl.BlockSpec((bn, d), lambda i: (i, 0)),
            pl.BlockSpec((bn, 1), lambda i: (i, 0)),
            pl.BlockSpec((1, d), lambda i: (0, 0)),
            pl.BlockSpec((1, 1), lambda i: (0, 0)),
        ],
        out_specs=pl.BlockSpec((bn, d), lambda i: (i, 0)),
        out_shape=jax.ShapeDtypeStruct((rows, d), jnp.float32),
    )(accp, g, dis, b.reshape(1, d), alpha.reshape(1, 1))

    return out[:n]
